# Initial kernel scaffold; baseline (speedup 1.0000x reference)
import functools

import jax
import jax.numpy as jnp
from jax import lax
from jax.experimental import pallas as pl
from jax.experimental.pallas import tpu as pltpu
from jax.experimental.pallas import tpu_sc as plsc

NC = 2
NS = 16
NW = NC * NS

K = 125
NCH = 80


@functools.cache
def _make_sc_segment_sum(NT, D):
    rows_per_tile = NT // NS
    mesh = plsc.VectorSubcoreMesh(core_axis_name="c", subcore_axis_name="s")

    @functools.partial(
        pl.kernel,
        out_type=jax.ShapeDtypeStruct((NC, NT, D), jnp.float32),
        mesh=mesh,
        scratch_types=[
            pltpu.VMEM((NCH, K), jnp.int32),
            pltpu.VMEM((NCH, K), jnp.int32),
            pltpu.VMEM((K, D), jnp.float32),
            pltpu.VMEM_SHARED((NT, D), jnp.float32),
            pltpu.SemaphoreType.DMA,
        ],
    )
    def sc_kernel(x0_hbm, src_hbm, dst_hbm, zeros_hbm, out_hbm,
                  src_v, dst_v, gbuf, acc, sem):
        c = lax.axis_index("c")
        s = lax.axis_index("s")
        wid = s * NC + c

        pltpu.sync_copy(src_hbm.at[wid], src_v)
        pltpu.sync_copy(dst_hbm.at[wid], dst_v)
        pltpu.sync_copy(zeros_hbm, acc.at[pl.ds(s * rows_per_tile, rows_per_tile)])
        plsc.subcore_barrier()

        @pl.loop(0, NCH)
        def _(j):
            pltpu.async_copy(x0_hbm.at[src_v.at[j]], gbuf, sem).wait()
            pltpu.sync_copy(gbuf, acc.at[dst_v.at[j]], add=True)

        plsc.subcore_barrier()
        pltpu.sync_copy(
            acc.at[pl.ds(s * rows_per_tile, rows_per_tile)],
            out_hbm.at[c, pl.ds(s * rows_per_tile, rows_per_tile)],
        )

    return sc_kernel


def _mlp_body(eps_ref, x1_ref, p0_ref, p1_ref, w1_ref, b1_ref, w2_ref, b2_ref,
              out_ref):
    h = (1.0 + eps_ref[0, 0]) * x1_ref[...] + p0_ref[...] + p1_ref[...]
    h = jnp.dot(h, w1_ref[...], preferred_element_type=jnp.float32) + b1_ref[...]
    h = jnp.maximum(h, 0.0)
    h = jnp.dot(h, w2_ref[...], preferred_element_type=jnp.float32) + b2_ref[...]
    out_ref[...] = jnp.maximum(h, 0.0)


@functools.cache
def _make_tc_mlp(NT, D, BLK):
    row_spec = pl.BlockSpec((BLK, D), lambda i: (i, 0))
    full_spec = pl.BlockSpec((D, D), lambda i: (0, 0))
    bias_spec = pl.BlockSpec((1, D), lambda i: (0, 0))
    return pl.pallas_call(
        _mlp_body,
        grid=(NT // BLK,),
        in_specs=[
            pl.BlockSpec(memory_space=pltpu.SMEM),
            row_spec, row_spec, row_spec,
            full_spec, bias_spec, full_spec, bias_spec,
        ],
        out_specs=row_spec,
        out_shape=jax.ShapeDtypeStruct((NT, D), jnp.float32),
    )


def kernel(x_0, x_1, edge_index_incidence_0_1, W1, b1, W2, b2, eps):
    NT, D = x_1.shape
    src = edge_index_incidence_0_1[0].astype(jnp.int32).reshape(NW, NCH, K)
    dst = edge_index_incidence_0_1[1].astype(jnp.int32).reshape(NW, NCH, K)
    zeros = jnp.zeros((NT // NS, D), jnp.float32)

    partials = _make_sc_segment_sum(NT, D)(x_0, src, dst, zeros)

    eps2 = jnp.reshape(eps, (1, 1)).astype(jnp.float32)
    out = _make_tc_mlp(NT, D, 2000)(
        eps2, x_1, partials[0], partials[1],
        W1, b1.reshape(1, D), W2, b2.reshape(1, D),
    )
    return out


# speedup vs baseline: 8.4309x; 8.4309x over previous
import functools

import jax
import jax.numpy as jnp
from jax import lax
from jax.experimental import pallas as pl
from jax.experimental.pallas import tpu as pltpu
from jax.experimental.pallas import tpu_sc as plsc

NC = 2
NS = 16
NW = NC * NS

K = 125
NCH = 80


@functools.cache
def _make_sc_segment_sum(NTP, D):
    rows_per_tile = NTP // NS
    mesh = plsc.VectorSubcoreMesh(core_axis_name="c", subcore_axis_name="s")

    @functools.partial(
        pl.kernel,
        out_type=jax.ShapeDtypeStruct((NC, NTP, D), jnp.float32),
        mesh=mesh,
        scratch_types=[
            pltpu.VMEM((NCH, K), jnp.int32),
            pltpu.VMEM((NCH, K), jnp.int32),
            pltpu.VMEM((K, D), jnp.float32),
            pltpu.VMEM_SHARED((NTP, D), jnp.float32),
            pltpu.SemaphoreType.DMA,
        ],
    )
    def sc_kernel(x0_hbm, src_hbm, dst_hbm, zeros_hbm, out_hbm,
                  src_v, dst_v, gbuf, acc, sem):
        c = lax.axis_index("c")
        s = lax.axis_index("s")
        wid = s * NC + c

        pltpu.sync_copy(src_hbm.at[wid], src_v)
        pltpu.sync_copy(dst_hbm.at[wid], dst_v)
        pltpu.sync_copy(zeros_hbm, acc.at[pl.ds(s * rows_per_tile, rows_per_tile)])
        plsc.subcore_barrier()

        @pl.loop(0, NCH)
        def _(j):
            pltpu.async_copy(x0_hbm.at[src_v.at[j]], gbuf, sem).wait()
            pltpu.sync_copy(gbuf, acc.at[dst_v.at[j]], add=True)

        plsc.subcore_barrier()
        pltpu.sync_copy(
            acc.at[pl.ds(s * rows_per_tile, rows_per_tile)],
            out_hbm.at[c, pl.ds(s * rows_per_tile, rows_per_tile)],
        )

    return sc_kernel


def _mlp_body(eps_ref, x1_ref, p0_ref, p1_ref, w1_ref, b1_ref, w2_ref, b2_ref,
              out_ref):
    h = (1.0 + eps_ref[0, 0]) * x1_ref[...] + p0_ref[...] + p1_ref[...]
    h = jnp.dot(h, w1_ref[...], preferred_element_type=jnp.float32) + b1_ref[...]
    h = jnp.maximum(h, 0.0)
    h = jnp.dot(h, w2_ref[...], preferred_element_type=jnp.float32) + b2_ref[...]
    out_ref[...] = jnp.maximum(h, 0.0)


@functools.cache
def _make_tc_mlp(NT, D, BLK):
    row_spec = pl.BlockSpec((BLK, D), lambda i: (i, 0))
    full_spec = pl.BlockSpec((D, D), lambda i: (0, 0))
    bias_spec = pl.BlockSpec((1, D), lambda i: (0, 0))
    return pl.pallas_call(
        _mlp_body,
        grid=(NT // BLK,),
        in_specs=[
            pl.BlockSpec(memory_space=pltpu.SMEM),
            row_spec, row_spec, row_spec,
            full_spec, bias_spec, full_spec, bias_spec,
        ],
        out_specs=row_spec,
        out_shape=jax.ShapeDtypeStruct((NT, D), jnp.float32),
    )


def kernel(x_0, x_1, edge_index_incidence_0_1, W1, b1, W2, b2, eps):
    NT, D = x_1.shape
    NTP = ((NT + NS * 8 - 1) // (NS * 8)) * NS * 8
    src = edge_index_incidence_0_1[0].astype(jnp.int32).reshape(NW, NCH, K)
    dst = edge_index_incidence_0_1[1].astype(jnp.int32).reshape(NW, NCH, K)
    zeros = jnp.zeros((NTP // NS, D), jnp.float32)

    partials = _make_sc_segment_sum(NTP, D)(x_0, src, dst, zeros)

    eps2 = jnp.reshape(eps, (1, 1)).astype(jnp.float32)
    out = _make_tc_mlp(NT, D, 2000)(
        eps2, x_1, partials[0], partials[1],
        W1, b1.reshape(1, D), W2, b2.reshape(1, D),
    )
    return out


# speedup vs baseline: 10.4306x; 1.2372x over previous
import functools

import jax
import jax.numpy as jnp
from jax import lax
from jax.experimental import pallas as pl
from jax.experimental.pallas import tpu as pltpu
from jax.experimental.pallas import tpu_sc as plsc

NC = 2
NS = 16
NW = NC * NS

K = 125
NCH = 80
NCH2 = NCH // 2


@functools.cache
def _make_sc_segment_sum(NTP, D):
    rows_per_tile = NTP // NS
    mesh = plsc.VectorSubcoreMesh(core_axis_name="c", subcore_axis_name="s")

    @functools.partial(
        pl.kernel,
        out_type=jax.ShapeDtypeStruct((NC, NTP, D), jnp.float32),
        mesh=mesh,
        scratch_types=[
            pltpu.VMEM((NCH2, K), jnp.int32),
            pltpu.VMEM((NCH2, K), jnp.int32),
            pltpu.VMEM((K, D), jnp.float32),
            pltpu.VMEM((K, D), jnp.float32),
            pltpu.VMEM_SHARED((NTP, D), jnp.float32),
            pltpu.SemaphoreType.DMA,
            pltpu.SemaphoreType.DMA,
        ],
    )
    def sc_kernel(x0_hbm, src_hbm, dst_hbm, zeros_hbm, out_hbm,
                  src_v, dst_v, gbuf0, gbuf1, acc, sem0, sem1):
        c = lax.axis_index("c")
        s = lax.axis_index("s")
        wid = s * NC + c

        pltpu.sync_copy(zeros_hbm, acc.at[pl.ds(s * rows_per_tile, rows_per_tile)])
        plsc.subcore_barrier()

        for h in range(2):
            pltpu.sync_copy(src_hbm.at[wid, pl.ds(h * NCH2, NCH2)], src_v)
            pltpu.sync_copy(dst_hbm.at[wid, pl.ds(h * NCH2, NCH2)], dst_v)
            pltpu.async_copy(x0_hbm.at[src_v.at[0]], gbuf0, sem0)

            @pl.loop(0, NCH2, step=2)
            def _(jj):
                pltpu.make_async_copy(x0_hbm.at[src_v.at[jj]], gbuf0, sem0).wait()
                pltpu.async_copy(x0_hbm.at[src_v.at[jj + 1]], gbuf1, sem1)
                pltpu.sync_copy(gbuf0, acc.at[dst_v.at[jj]], add=True)

                pltpu.make_async_copy(x0_hbm.at[src_v.at[jj]], gbuf1, sem1).wait()

                @pl.when(jj + 2 < NCH2)
                def _():
                    pltpu.async_copy(x0_hbm.at[src_v.at[jj + 2]], gbuf0, sem0)

                pltpu.sync_copy(gbuf1, acc.at[dst_v.at[jj + 1]], add=True)

        plsc.subcore_barrier()
        pltpu.sync_copy(
            acc.at[pl.ds(s * rows_per_tile, rows_per_tile)],
            out_hbm.at[c, pl.ds(s * rows_per_tile, rows_per_tile)],
        )

    return sc_kernel


def _mlp_body(eps_ref, x1_ref, p0_ref, p1_ref, w1_ref, b1_ref, w2_ref, b2_ref,
              out_ref):
    h = (1.0 + eps_ref[0, 0]) * x1_ref[...] + p0_ref[...] + p1_ref[...]
    h = jnp.dot(h, w1_ref[...], preferred_element_type=jnp.float32) + b1_ref[...]
    h = jnp.maximum(h, 0.0)
    h = jnp.dot(h, w2_ref[...], preferred_element_type=jnp.float32) + b2_ref[...]
    out_ref[...] = jnp.maximum(h, 0.0)


@functools.cache
def _make_tc_mlp(NT, D, BLK):
    row_spec = pl.BlockSpec((BLK, D), lambda i: (i, 0))
    full_spec = pl.BlockSpec((D, D), lambda i: (0, 0))
    bias_spec = pl.BlockSpec((1, D), lambda i: (0, 0))
    return pl.pallas_call(
        _mlp_body,
        grid=(NT // BLK,),
        in_specs=[
            pl.BlockSpec(memory_space=pltpu.SMEM),
            row_spec, row_spec, row_spec,
            full_spec, bias_spec, full_spec, bias_spec,
        ],
        out_specs=row_spec,
        out_shape=jax.ShapeDtypeStruct((NT, D), jnp.float32),
    )


def kernel(x_0, x_1, edge_index_incidence_0_1, W1, b1, W2, b2, eps):
    NT, D = x_1.shape
    NTP = ((NT + NS * 8 - 1) // (NS * 8)) * NS * 8
    src = edge_index_incidence_0_1[0].astype(jnp.int32).reshape(NW, NCH, K)
    dst = edge_index_incidence_0_1[1].astype(jnp.int32).reshape(NW, NCH, K)
    zeros = jnp.zeros((NTP // NS, D), jnp.float32)

    partials = _make_sc_segment_sum(NTP, D)(x_0, src, dst, zeros)

    eps2 = jnp.reshape(eps, (1, 1)).astype(jnp.float32)
    out = _make_tc_mlp(NT, D, 2000)(
        eps2, x_1, partials[0], partials[1],
        W1, b1.reshape(1, D), W2, b2.reshape(1, D),
    )
    return out


# speedup vs baseline: 10.4331x; 1.0002x over previous
import functools

import jax
import jax.numpy as jnp
from jax import lax
from jax.experimental import pallas as pl
from jax.experimental.pallas import tpu as pltpu
from jax.experimental.pallas import tpu_sc as plsc

NC = 2
NS = 16
NW = NC * NS

K = 125
NCH = 80
NCH2 = NCH // 2


@functools.cache
def _make_sc_segment_sum(NTP, D):
    rows_per_tile = NTP // NS
    mesh = plsc.VectorSubcoreMesh(core_axis_name="c", subcore_axis_name="s")

    @functools.partial(
        pl.kernel,
        out_type=jax.ShapeDtypeStruct((NC, NTP, D), jnp.float32),
        mesh=mesh,
        scratch_types=[
            pltpu.VMEM((NCH2, K), jnp.int32),
            pltpu.VMEM((NCH2, K), jnp.int32),
            pltpu.VMEM((K, D), jnp.float32),
            pltpu.VMEM((K, D), jnp.float32),
            pltpu.VMEM_SHARED((NTP, D), jnp.float32),
            pltpu.SemaphoreType.DMA,
            pltpu.SemaphoreType.DMA,
            pltpu.SemaphoreType.DMA,
            pltpu.SemaphoreType.DMA,
        ],
    )
    def sc_kernel(x0_hbm, src_hbm, dst_hbm, zeros_hbm, out_hbm,
                  src_v, dst_v, gbuf0, gbuf1, acc, sem0, sem1, ssem0, ssem1):
        c = lax.axis_index("c")
        s = lax.axis_index("s")
        wid = s * NC + c

        pltpu.sync_copy(zeros_hbm, acc.at[pl.ds(s * rows_per_tile, rows_per_tile)])
        plsc.subcore_barrier()

        for h in range(2):
            pltpu.sync_copy(src_hbm.at[wid, pl.ds(h * NCH2, NCH2)], src_v)
            pltpu.sync_copy(dst_hbm.at[wid, pl.ds(h * NCH2, NCH2)], dst_v)
            pltpu.async_copy(x0_hbm.at[src_v.at[0]], gbuf0, sem0)

            @pl.loop(0, NCH2, step=2)
            def _(jj):
                pltpu.make_async_copy(x0_hbm.at[src_v.at[jj]], gbuf0, sem0).wait()
                pltpu.async_copy(gbuf0, acc.at[dst_v.at[jj]], ssem0, add=True)

                @pl.when(jj > 0)
                def _():
                    pltpu.make_async_copy(gbuf1, acc.at[dst_v.at[jj]], ssem1).wait()

                pltpu.async_copy(x0_hbm.at[src_v.at[jj + 1]], gbuf1, sem1)
                pltpu.make_async_copy(x0_hbm.at[src_v.at[jj]], gbuf1, sem1).wait()
                pltpu.async_copy(gbuf1, acc.at[dst_v.at[jj + 1]], ssem1, add=True)

                pltpu.make_async_copy(gbuf0, acc.at[dst_v.at[jj]], ssem0).wait()

                @pl.when(jj + 2 < NCH2)
                def _():
                    pltpu.async_copy(x0_hbm.at[src_v.at[jj + 2]], gbuf0, sem0)

            pltpu.make_async_copy(gbuf1, acc.at[dst_v.at[0]], ssem1).wait()

        plsc.subcore_barrier()
        pltpu.sync_copy(
            acc.at[pl.ds(s * rows_per_tile, rows_per_tile)],
            out_hbm.at[c, pl.ds(s * rows_per_tile, rows_per_tile)],
        )

    return sc_kernel


def _mlp_body(eps_ref, x1_ref, p0_ref, p1_ref, w1_ref, b1_ref, w2_ref, b2_ref,
              out_ref):
    h = (1.0 + eps_ref[0, 0]) * x1_ref[...] + p0_ref[...] + p1_ref[...]
    h = jnp.dot(h, w1_ref[...], preferred_element_type=jnp.float32) + b1_ref[...]
    h = jnp.maximum(h, 0.0)
    h = jnp.dot(h, w2_ref[...], preferred_element_type=jnp.float32) + b2_ref[...]
    out_ref[...] = jnp.maximum(h, 0.0)


@functools.cache
def _make_tc_mlp(NT, D, BLK):
    row_spec = pl.BlockSpec((BLK, D), lambda i: (i, 0))
    full_spec = pl.BlockSpec((D, D), lambda i: (0, 0))
    bias_spec = pl.BlockSpec((1, D), lambda i: (0, 0))
    return pl.pallas_call(
        _mlp_body,
        grid=(NT // BLK,),
        in_specs=[
            pl.BlockSpec(memory_space=pltpu.SMEM),
            row_spec, row_spec, row_spec,
            full_spec, bias_spec, full_spec, bias_spec,
        ],
        out_specs=row_spec,
        out_shape=jax.ShapeDtypeStruct((NT, D), jnp.float32),
    )


def kernel(x_0, x_1, edge_index_incidence_0_1, W1, b1, W2, b2, eps):
    NT, D = x_1.shape
    NTP = ((NT + NS * 8 - 1) // (NS * 8)) * NS * 8
    src = edge_index_incidence_0_1[0].astype(jnp.int32).reshape(NW, NCH, K)
    dst = edge_index_incidence_0_1[1].astype(jnp.int32).reshape(NW, NCH, K)
    zeros = jnp.zeros((NTP // NS, D), jnp.float32)

    partials = _make_sc_segment_sum(NTP, D)(x_0, src, dst, zeros)

    eps2 = jnp.reshape(eps, (1, 1)).astype(jnp.float32)
    out = _make_tc_mlp(NT, D, 2000)(
        eps2, x_1, partials[0], partials[1],
        W1, b1.reshape(1, D), W2, b2.reshape(1, D),
    )
    return out


# speedup vs baseline: 10.8609x; 1.0410x over previous
import functools

import jax
import jax.numpy as jnp
from jax import lax
from jax.experimental import pallas as pl
from jax.experimental.pallas import tpu as pltpu
from jax.experimental.pallas import tpu_sc as plsc

NC = 2
NS = 16
NW = NC * NS

K = 125
NCH = 80
NCH2 = NCH // 2


@functools.cache
def _make_sc_segment_sum(NTP, D):
    rows_per_tile = NTP // NS
    mesh = plsc.VectorSubcoreMesh(core_axis_name="c", subcore_axis_name="s")

    @functools.partial(
        pl.kernel,
        out_type=jax.ShapeDtypeStruct((NC, NTP, D), jnp.float32),
        mesh=mesh,
        scratch_types=[
            pltpu.VMEM((NCH2, K), jnp.int32),
            pltpu.VMEM((NCH2, K), jnp.int32),
            pltpu.VMEM((K, D), jnp.float32),
            pltpu.VMEM((K, D), jnp.float32),
            pltpu.VMEM_SHARED((NTP, D), jnp.float32),
            pltpu.SemaphoreType.DMA,
            pltpu.SemaphoreType.DMA,
            pltpu.SemaphoreType.DMA,
            pltpu.SemaphoreType.DMA,
        ],
    )
    def sc_kernel(x0_hbm, src_hbm, dst_hbm, zeros_hbm, out_hbm,
                  src_v, dst_v, gbuf0, gbuf1, acc, sem0, sem1, ssem0, ssem1):
        c = lax.axis_index("c")
        s = lax.axis_index("s")
        wid = s * NC + c

        pltpu.sync_copy(zeros_hbm, acc.at[pl.ds(s * rows_per_tile, rows_per_tile)])
        plsc.subcore_barrier()

        for h in range(2):
            pltpu.sync_copy(src_hbm.at[wid, pl.ds(h * NCH2, NCH2)], src_v)
            pltpu.sync_copy(dst_hbm.at[wid, pl.ds(h * NCH2, NCH2)], dst_v)
            pltpu.async_copy(x0_hbm.at[src_v.at[0]], gbuf0, sem0)

            @pl.loop(0, NCH2, step=2)
            def _(jj):
                pltpu.make_async_copy(x0_hbm.at[src_v.at[jj]], gbuf0, sem0).wait()
                pltpu.async_copy(gbuf0, acc.at[dst_v.at[jj]], ssem0, add=True)

                @pl.when(jj > 0)
                def _():
                    pltpu.make_async_copy(gbuf1, acc.at[dst_v.at[jj]], ssem1).wait()

                pltpu.async_copy(x0_hbm.at[src_v.at[jj + 1]], gbuf1, sem1)
                pltpu.make_async_copy(x0_hbm.at[src_v.at[jj]], gbuf1, sem1).wait()
                pltpu.async_copy(gbuf1, acc.at[dst_v.at[jj + 1]], ssem1, add=True)

                pltpu.make_async_copy(gbuf0, acc.at[dst_v.at[jj]], ssem0).wait()

                @pl.when(jj + 2 < NCH2)
                def _():
                    pltpu.async_copy(x0_hbm.at[src_v.at[jj + 2]], gbuf0, sem0)

            pltpu.make_async_copy(gbuf1, acc.at[dst_v.at[0]], ssem1).wait()

        plsc.subcore_barrier()
        pltpu.sync_copy(
            acc.at[pl.ds(s * rows_per_tile, rows_per_tile)],
            out_hbm.at[c, pl.ds(s * rows_per_tile, rows_per_tile)],
        )

    return sc_kernel


def _mlp_body(eps_ref, x1_ref, p0_ref, p1_ref, w1_ref, b1_ref, w2_ref, b2_ref,
              out_ref):
    h = (1.0 + eps_ref[0, 0]) * x1_ref[...] + p0_ref[0] + p1_ref[0]
    h = jnp.dot(h, w1_ref[...], preferred_element_type=jnp.float32) + b1_ref[...]
    h = jnp.maximum(h, 0.0)
    h = jnp.dot(h, w2_ref[...], preferred_element_type=jnp.float32) + b2_ref[...]
    out_ref[...] = jnp.maximum(h, 0.0)


@functools.cache
def _make_tc_mlp(NT, D, BLK):
    row_spec = pl.BlockSpec((BLK, D), lambda i: (i, 0))
    p0_spec = pl.BlockSpec((1, BLK, D), lambda i: (0, i, 0))
    p1_spec = pl.BlockSpec((1, BLK, D), lambda i: (1, i, 0))
    full_spec = pl.BlockSpec((D, D), lambda i: (0, 0))
    bias_spec = pl.BlockSpec((1, D), lambda i: (0, 0))
    return pl.pallas_call(
        _mlp_body,
        grid=(NT // BLK,),
        in_specs=[
            pl.BlockSpec(memory_space=pltpu.SMEM),
            row_spec, p0_spec, p1_spec,
            full_spec, bias_spec, full_spec, bias_spec,
        ],
        out_specs=row_spec,
        out_shape=jax.ShapeDtypeStruct((NT, D), jnp.float32),
    )


def kernel(x_0, x_1, edge_index_incidence_0_1, W1, b1, W2, b2, eps):
    NT, D = x_1.shape
    NTP = ((NT + NS * 8 - 1) // (NS * 8)) * NS * 8
    src = edge_index_incidence_0_1[0].astype(jnp.int32).reshape(NW, NCH, K)
    dst = edge_index_incidence_0_1[1].astype(jnp.int32).reshape(NW, NCH, K)
    zeros = jnp.zeros((NTP // NS, D), jnp.float32)

    partials = _make_sc_segment_sum(NTP, D)(x_0, src, dst, zeros)

    eps2 = jnp.reshape(eps, (1, 1)).astype(jnp.float32)
    out = _make_tc_mlp(NT, D, 2000)(
        eps2, x_1, partials, partials,
        W1, b1.reshape(1, D), W2, b2.reshape(1, D),
    )
    return out


# speedup vs baseline: 11.5637x; 1.0647x over previous
import functools

import jax
import jax.numpy as jnp
from jax import lax
from jax.experimental import pallas as pl
from jax.experimental.pallas import tpu as pltpu
from jax.experimental.pallas import tpu_sc as plsc

NC = 2
NS = 16
NW = NC * NS

K = 125
NCH = 80
NCH2 = NCH // 2


@functools.cache
def _make_sc_segment_sum(NTP, D):
    rows_per_tile = NTP // NS
    mesh = plsc.VectorSubcoreMesh(core_axis_name="c", subcore_axis_name="s")

    @functools.partial(
        pl.kernel,
        out_type=jax.ShapeDtypeStruct((NC, NTP, D), jnp.float32),
        mesh=mesh,
        scratch_types=[
            pltpu.VMEM((NCH2, K), jnp.int32),
            pltpu.VMEM((NCH2, K), jnp.int32),
            pltpu.VMEM((K, D), jnp.float32),
            pltpu.VMEM((K, D), jnp.float32),
            pltpu.VMEM_SHARED((NTP, D), jnp.float32),
            pltpu.SemaphoreType.DMA,
            pltpu.SemaphoreType.DMA,
            pltpu.SemaphoreType.DMA,
            pltpu.SemaphoreType.DMA,
        ],
    )
    def sc_kernel(x0_hbm, eidx_hbm, zeros_hbm, out_hbm,
                  src_v, dst_v, gbuf0, gbuf1, acc, sem0, sem1, ssem0, ssem1):
        c = lax.axis_index("c")
        s = lax.axis_index("s")
        wid = s * NC + c

        pltpu.sync_copy(zeros_hbm, acc.at[pl.ds(s * rows_per_tile, rows_per_tile)])
        plsc.subcore_barrier()

        for h in range(2):
            pltpu.sync_copy(eidx_hbm.at[0, wid, pl.ds(h * NCH2, NCH2)], src_v)
            pltpu.sync_copy(eidx_hbm.at[1, wid, pl.ds(h * NCH2, NCH2)], dst_v)
            pltpu.async_copy(x0_hbm.at[src_v.at[0]], gbuf0, sem0)

            @pl.loop(0, NCH2, step=2)
            def _(jj):
                pltpu.make_async_copy(x0_hbm.at[src_v.at[jj]], gbuf0, sem0).wait()
                pltpu.async_copy(gbuf0, acc.at[dst_v.at[jj]], ssem0, add=True)

                @pl.when(jj > 0)
                def _():
                    pltpu.make_async_copy(gbuf1, acc.at[dst_v.at[jj]], ssem1).wait()

                pltpu.async_copy(x0_hbm.at[src_v.at[jj + 1]], gbuf1, sem1)
                pltpu.make_async_copy(x0_hbm.at[src_v.at[jj]], gbuf1, sem1).wait()
                pltpu.async_copy(gbuf1, acc.at[dst_v.at[jj + 1]], ssem1, add=True)

                pltpu.make_async_copy(gbuf0, acc.at[dst_v.at[jj]], ssem0).wait()

                @pl.when(jj + 2 < NCH2)
                def _():
                    pltpu.async_copy(x0_hbm.at[src_v.at[jj + 2]], gbuf0, sem0)

            pltpu.make_async_copy(gbuf1, acc.at[dst_v.at[0]], ssem1).wait()

        plsc.subcore_barrier()
        pltpu.sync_copy(
            acc.at[pl.ds(s * rows_per_tile, rows_per_tile)],
            out_hbm.at[c, pl.ds(s * rows_per_tile, rows_per_tile)],
        )

    return sc_kernel


def _mlp_body(eps_ref, x1_ref, p0_ref, p1_ref, w1_ref, b1_ref, w2_ref, b2_ref,
              out_ref):
    h = (1.0 + eps_ref[0, 0]) * x1_ref[...] + p0_ref[0] + p1_ref[0]
    h = jnp.dot(h, w1_ref[...], preferred_element_type=jnp.float32) + b1_ref[...]
    h = jnp.maximum(h, 0.0)
    h = jnp.dot(h, w2_ref[...], preferred_element_type=jnp.float32) + b2_ref[...]
    out_ref[...] = jnp.maximum(h, 0.0)


@functools.cache
def _make_tc_mlp(NT, D, BLK):
    row_spec = pl.BlockSpec((BLK, D), lambda i: (i, 0))
    p0_spec = pl.BlockSpec((1, BLK, D), lambda i: (0, i, 0))
    p1_spec = pl.BlockSpec((1, BLK, D), lambda i: (1, i, 0))
    full_spec = pl.BlockSpec((D, D), lambda i: (0, 0))
    bias_spec = pl.BlockSpec((1, D), lambda i: (0, 0))
    return pl.pallas_call(
        _mlp_body,
        grid=(NT // BLK,),
        in_specs=[
            pl.BlockSpec(memory_space=pltpu.SMEM),
            row_spec, p0_spec, p1_spec,
            full_spec, bias_spec, full_spec, bias_spec,
        ],
        out_specs=row_spec,
        out_shape=jax.ShapeDtypeStruct((NT, D), jnp.float32),
    )


def kernel(x_0, x_1, edge_index_incidence_0_1, W1, b1, W2, b2, eps):
    NT, D = x_1.shape
    NTP = ((NT + NS * 8 - 1) // (NS * 8)) * NS * 8
    eidx = edge_index_incidence_0_1.astype(jnp.int32).reshape(2, NW, NCH, K)
    zeros = jnp.zeros((NTP // NS, D), jnp.float32)

    partials = _make_sc_segment_sum(NTP, D)(x_0, eidx, zeros)

    eps2 = jnp.reshape(eps, (1, 1)).astype(jnp.float32)
    out = _make_tc_mlp(NT, D, 2000)(
        eps2, x_1, partials, partials,
        W1, b1.reshape(1, D), W2, b2.reshape(1, D),
    )
    return out
